# baseline (speedup 1.0000x reference)
import functools

import jax
import jax.numpy as jnp
from jax import lax
from jax.experimental import pallas as pl
from jax.experimental.pallas import tpu as pltpu
from jax.experimental.pallas import tpu_sc as plsc

_IDX_CHUNK = 128


def _sc_gather(table, idx):
    B = idx.shape[0]
    D = table.shape[1]
    V = table.shape[0]
    pairs = table.reshape(V // 2, 2 * D)
    pair_idx = lax.shift_right_logical(idx, 1)
    info = plsc.get_sparse_core_info()
    nc, ns = info.num_cores, info.num_subcores
    nw = nc * ns
    b_per_w = B // nw
    n_chunks = b_per_w // _IDX_CHUNK
    mesh = plsc.VectorSubcoreMesh(core_axis_name="c", subcore_axis_name="s")

    @functools.partial(
        pl.kernel,
        mesh=mesh,
        out_type=jax.ShapeDtypeStruct((B, 2 * D), jnp.float32),
        scratch_types=[
            pltpu.VMEM((b_per_w,), jnp.int32),
            pltpu.VMEM((b_per_w, 2 * D), jnp.float32),
            pltpu.SemaphoreType.DMA,
        ],
    )
    def k(table_hbm, idx_hbm, out_hbm, idx_v, rows_v, sem):
        wid = lax.axis_index("s") * nc + lax.axis_index("c")
        base = wid * b_per_w
        pltpu.sync_copy(idx_hbm.at[pl.ds(base, b_per_w)], idx_v)
        for g in range(n_chunks):
            pltpu.make_async_copy(
                table_hbm.at[idx_v.at[pl.ds(g * _IDX_CHUNK, _IDX_CHUNK)]],
                rows_v.at[pl.ds(g * _IDX_CHUNK, _IDX_CHUNK)],
                sem,
            ).start()
        pltpu.make_async_copy(
            table_hbm.at[pl.ds(0, b_per_w)], rows_v, sem
        ).wait()
        pltpu.sync_copy(rows_v, out_hbm.at[pl.ds(base, b_per_w)])

    return k(pairs, pair_idx)


def _tc_dense(pairs, parity, mean, W1, b1):
    B = pairs.shape[0]
    D = mean.shape[1]
    blk = 2048

    def body(e_ref, p_ref, m_ref, w_ref, b_ref, o_ref):
        x = jnp.where(p_ref[...] > 0, e_ref[:, D:], e_ref[:, :D])
        x = x + m_ref[...]
        y = lax.dot_general(
            x, w_ref[...], (((1,), (1,)), ((), ())),
            preferred_element_type=jnp.float32,
        )
        y = y + b_ref[...]
        o_ref[...] = jnp.where(y >= 0, y, 0.2 * y)

    return pl.pallas_call(
        body,
        grid=(B // blk,),
        in_specs=[
            pl.BlockSpec((blk, 2 * D), lambda i: (i, 0)),
            pl.BlockSpec((blk, 1), lambda i: (i, 0)),
            pl.BlockSpec((blk, D), lambda i: (i, 0)),
            pl.BlockSpec((D, D), lambda i: (0, 0)),
            pl.BlockSpec((1, D), lambda i: (0, 0)),
        ],
        out_specs=pl.BlockSpec((blk, D), lambda i: (i, 0)),
        out_shape=jax.ShapeDtypeStruct((B, D), jnp.float32),
    )(pairs, parity, mean, W1, b1.reshape(1, D))


def kernel(user_idx, mean_poi_embeddings, user_embedding, W1, b1):
    idx = user_idx.astype(jnp.int32)
    pairs = _sc_gather(user_embedding, idx)
    parity = (idx & 1).reshape(-1, 1)
    return _tc_dense(pairs, parity, mean_poi_embeddings, W1, b1)
